# baseline (speedup 1.0000x reference)
import functools

import jax
import jax.numpy as jnp
from jax import lax
from jax.experimental import pallas as pl
from jax.experimental.pallas import tpu as pltpu
from jax.experimental.pallas import tpu_sc as plsc

_D = 64
_BB = 128
_NC = 2
_NS = 16
_NW = _NC * _NS
_NBUF = 4
_AHEAD = 3


@functools.lru_cache(maxsize=None)
def _make_kernel(batch, hist):
    assert batch == _BB * _NW
    assert hist % _NBUF == 0
    mesh = plsc.VectorSubcoreMesh(core_axis_name="c", subcore_axis_name="s")

    @functools.partial(
        pl.kernel,
        out_type=jax.ShapeDtypeStruct(
            (hist, _D // 8, batch // 128, 8, 128), jnp.float32
        ),
        mesh=mesh,
        scratch_types=[
            pltpu.VMEM((hist, _BB), jnp.int32),
            pltpu.VMEM((_NBUF, _BB, _D), jnp.float32),
            pltpu.VMEM((_NBUF, _D // 8, 8, 128), jnp.float32),
            pltpu.SemaphoreType.DMA((_NBUF,)),
            pltpu.SemaphoreType.DMA((_NBUF,)),
            pltpu.SemaphoreType.DMA,
        ],
        compiler_params=pltpu.CompilerParams(
            use_tc_tiling_on_sc=False, needs_layout_passes=False
        ),
    )
    def body(wt_hbm, table_hbm, out_hbm, idx_v, rows_v, t_v, g_sem, o_sem,
             i_sem):
        wid = lax.axis_index("s") * _NC + lax.axis_index("c")
        pltpu.async_copy(
            wt_hbm.at[:, pl.ds(wid * _BB, _BB)], idx_v, i_sem
        ).wait()

        def start_gather(h, slot):
            pltpu.async_copy(
                table_hbm.at[idx_v.at[h]], rows_v.at[slot], g_sem.at[slot]
            )

        for p in range(_AHEAD):
            start_gather(p, p)

        bvecs = [lax.iota(jnp.int32, 16) + 16 * k for k in range(_BB // 16)]

        def group(g, carry):
            for p in range(_NBUF):
                h = g * _NBUF + p

                pltpu.make_async_copy(
                    table_hbm.at[idx_v.at[h]],
                    rows_v.at[p],
                    g_sem.at[p],
                ).wait()

                @pl.when(h >= _NBUF)
                def _():
                    pltpu.make_async_copy(
                        t_v.at[p],
                        out_hbm.at[0, :, 0],
                        o_sem.at[p],
                    ).wait()

                rows = rows_v.at[p]
                for d in range(_D):
                    dvec = jnp.full((16,), d, jnp.int32)
                    for k in range(_BB // 16):
                        vals = plsc.load_gather(rows, [bvecs[k], dvec])
                        t_v[p, d // 8, d % 8, pl.ds(16 * k, 16)] = vals

                pltpu.async_copy(
                    t_v.at[p],
                    out_hbm.at[h, :, wid],
                    o_sem.at[p],
                )

                h2 = h + _AHEAD

                @pl.when(h2 < hist)
                def _():
                    start_gather(h2, (p + _AHEAD) % _NBUF)
            return carry

        lax.fori_loop(0, hist // _NBUF, group, 0)

        for p in range(_NBUF):
            pltpu.make_async_copy(
                t_v.at[p], out_hbm.at[0, :, 0], o_sem.at[p]
            ).wait()

    return body


def kernel(words, table):
    b, h = words.shape
    out5 = _make_kernel(b, h)(words.T, table)
    return out5.transpose(2, 4, 0, 1, 3).reshape(b, h, _D)


# speedup vs baseline: 1.1161x; 1.1161x over previous
import functools

import jax
import jax.numpy as jnp
from jax import lax
from jax.experimental import pallas as pl
from jax.experimental.pallas import tpu as pltpu
from jax.experimental.pallas import tpu_sc as plsc

_D = 64
_CHUNK = 128
_NC = 2
_NS = 16
_NW = _NC * _NS
_NBUF = 8
_AHEAD = 4


@functools.lru_cache(maxsize=None)
def _make_gather(n_total):
    n_per_w = n_total // _NW
    n_chunks = n_per_w // _CHUNK
    assert n_chunks % _NBUF == 0
    mesh = plsc.VectorSubcoreMesh(core_axis_name="c", subcore_axis_name="s")

    @functools.partial(
        pl.kernel,
        out_type=jax.ShapeDtypeStruct((n_total, _D), jnp.float32),
        mesh=mesh,
        scratch_types=[
            pltpu.VMEM((n_chunks, _CHUNK), jnp.int32),
            pltpu.VMEM((_NBUF, _CHUNK, _D), jnp.float32),
            pltpu.SemaphoreType.DMA((_NBUF,)),
            pltpu.SemaphoreType.DMA((_NBUF,)),
        ],
        compiler_params=pltpu.CompilerParams(use_tc_tiling_on_sc=False),
    )
    def body(idx_hbm, table_hbm, out_hbm, idx_v, rows_v, g_sem, o_sem):
        wid = lax.axis_index("s") * _NC + lax.axis_index("c")
        pltpu.sync_copy(idx_hbm.at[wid], idx_v)
        base = wid * n_per_w

        def start_gather(j, slot):
            pltpu.async_copy(
                table_hbm.at[idx_v.at[j]], rows_v.at[slot], g_sem.at[slot]
            )

        for b in range(_AHEAD):
            start_gather(b, b)

        def group(g, carry):
            for b in range(_NBUF):
                j = g * _NBUF + b
                j2 = j + _AHEAD
                s2 = (b + _AHEAD) % _NBUF

                @pl.when(jnp.logical_and(j2 < n_chunks, j2 >= _NBUF))
                def _():
                    pltpu.make_async_copy(
                        rows_v.at[s2],
                        out_hbm.at[pl.ds(base, _CHUNK)],
                        o_sem.at[s2],
                    ).wait()

                @pl.when(j2 < n_chunks)
                def _():
                    start_gather(j2, s2)

                pltpu.make_async_copy(
                    table_hbm.at[idx_v.at[j]],
                    rows_v.at[b],
                    g_sem.at[b],
                ).wait()
                pltpu.async_copy(
                    rows_v.at[b],
                    out_hbm.at[pl.ds(base + j * _CHUNK, _CHUNK)],
                    o_sem.at[b],
                )
            return carry

        lax.fori_loop(0, n_chunks // _NBUF, group, 0)

        for b in range(_NBUF):
            pltpu.make_async_copy(
                rows_v.at[b], out_hbm.at[pl.ds(base, _CHUNK)], o_sem.at[b]
            ).wait()

    return body


def _relayout_body(flat_ref, out_ref):
    x = flat_ref[...].reshape(_CHUNK, 8 * _D)
    eye = (
        lax.broadcasted_iota(jnp.int32, (_CHUNK, _CHUNK), 0)
        == lax.broadcasted_iota(jnp.int32, (_CHUNK, _CHUNK), 1)
    ).astype(jnp.float32)
    xt = lax.dot_general(
        x, eye, (((0,), (0,)), ((), ())),
        preferred_element_type=jnp.float32,
    )
    out_ref[...] = xt.reshape(8, 8, 1, 8, _CHUNK)


@functools.lru_cache(maxsize=None)
def _make_relayout(batch, hist):
    grid = (batch // _CHUNK, hist // 8)
    return pl.pallas_call(
        _relayout_body,
        grid=grid,
        in_specs=[
            pl.BlockSpec((_CHUNK, 8, _D), lambda i, h: (i, h, 0)),
        ],
        out_specs=pl.BlockSpec(
            (8, 8, 1, 8, _CHUNK), lambda i, h: (h, 0, i, 0, 0)
        ),
        out_shape=jax.ShapeDtypeStruct(
            (hist, _D // 8, batch // _CHUNK, 8, _CHUNK), jnp.float32
        ),
    )


def kernel(words, table):
    b, h = words.shape
    n_total = b * h
    idx = words.reshape(_NW, n_total // _NW // _CHUNK, _CHUNK)
    flat = _make_gather(n_total)(idx, table)
    out5 = _make_relayout(b, h)(flat.reshape(b, h, _D))
    return out5.transpose(2, 4, 0, 1, 3).reshape(b, h, _D)


# speedup vs baseline: 1.3629x; 1.2211x over previous
import functools

import jax
import jax.numpy as jnp
from jax import lax
from jax.experimental import pallas as pl
from jax.experimental.pallas import tpu as pltpu
from jax.experimental.pallas import tpu_sc as plsc

_D = 64
_CHUNK = 128
_NC = 2
_NS = 16
_NW = _NC * _NS
_NBUF = 8
_AHEAD = 4


@functools.lru_cache(maxsize=None)
def _make_gather(n_total):
    n_per_w = n_total // _NW
    n_chunks = n_per_w // _CHUNK
    assert n_chunks % _NBUF == 0
    mesh = plsc.VectorSubcoreMesh(core_axis_name="c", subcore_axis_name="s")

    @functools.partial(
        pl.kernel,
        out_type=jax.ShapeDtypeStruct((n_total, _D), jnp.float32),
        mesh=mesh,
        scratch_types=[
            pltpu.VMEM((n_chunks, _CHUNK), jnp.int32),
            pltpu.VMEM((_NBUF, _CHUNK, _D), jnp.float32),
            pltpu.SemaphoreType.DMA((_NBUF,)),
            pltpu.SemaphoreType.DMA((_NBUF,)),
        ],
        compiler_params=pltpu.CompilerParams(use_tc_tiling_on_sc=False),
    )
    def body(idx_hbm, table_hbm, out_hbm, idx_v, rows_v, g_sem, o_sem):
        wid = lax.axis_index("s") * _NC + lax.axis_index("c")
        pltpu.sync_copy(idx_hbm.at[wid], idx_v)
        base = wid * n_per_w

        def start_gather(j, slot):
            pltpu.async_copy(
                table_hbm.at[idx_v.at[j]], rows_v.at[slot], g_sem.at[slot]
            )

        for b in range(_AHEAD):
            start_gather(b, b)

        def group(g, carry):
            for b in range(_NBUF):
                j = g * _NBUF + b
                j2 = j + _AHEAD
                s2 = (b + _AHEAD) % _NBUF

                @pl.when(jnp.logical_and(j2 < n_chunks, j2 >= _NBUF))
                def _():
                    pltpu.make_async_copy(
                        rows_v.at[s2],
                        out_hbm.at[pl.ds(base, _CHUNK)],
                        o_sem.at[s2],
                    ).wait()

                @pl.when(j2 < n_chunks)
                def _():
                    start_gather(j2, s2)

                pltpu.make_async_copy(
                    table_hbm.at[idx_v.at[j]],
                    rows_v.at[b],
                    g_sem.at[b],
                ).wait()
                pltpu.async_copy(
                    rows_v.at[b],
                    out_hbm.at[pl.ds(base + j * _CHUNK, _CHUNK)],
                    o_sem.at[b],
                )
            return carry

        lax.fori_loop(0, n_chunks // _NBUF, group, 0)

        for b in range(_NBUF):
            pltpu.make_async_copy(
                rows_v.at[b], out_hbm.at[pl.ds(base, _CHUNK)], o_sem.at[b]
            ).wait()

    return body


def _make_relayout_body(hb):
    def body(flat_ref, out_ref):
        x = flat_ref[...].reshape(_CHUNK, hb * _D)
        eye = (
            lax.broadcasted_iota(jnp.int32, (_CHUNK, _CHUNK), 0)
            == lax.broadcasted_iota(jnp.int32, (_CHUNK, _CHUNK), 1)
        ).astype(jnp.float32)
        xt = lax.dot_general(
            x, eye, (((0,), (0,)), ((), ())),
            precision=lax.Precision.HIGHEST,
            preferred_element_type=jnp.float32,
        )
        out_ref[...] = xt.reshape(hb, 8, 1, 8, _CHUNK)

    return body


@functools.lru_cache(maxsize=None)
def _make_relayout(batch, hist):
    hb = 40
    grid = (batch // _CHUNK, hist // hb)
    return pl.pallas_call(
        _make_relayout_body(hb),
        grid=grid,
        in_specs=[
            pl.BlockSpec((_CHUNK, hb, _D), lambda i, h: (i, h, 0)),
        ],
        out_specs=pl.BlockSpec(
            (hb, 8, 1, 8, _CHUNK), lambda i, h: (h, 0, i, 0, 0)
        ),
        out_shape=jax.ShapeDtypeStruct(
            (hist, _D // 8, batch // _CHUNK, 8, _CHUNK), jnp.float32
        ),
    )


def kernel(words, table):
    b, h = words.shape
    n_total = b * h
    idx = words.reshape(_NW, n_total // _NW // _CHUNK, _CHUNK)
    flat = _make_gather(n_total)(idx, table)
    out5 = _make_relayout(b, h)(flat.reshape(b, h, _D))
    return out5.transpose(2, 4, 0, 1, 3).reshape(b, h, _D)


# speedup vs baseline: 1.4355x; 1.0533x over previous
import functools

import jax
import jax.numpy as jnp
from jax import lax
from jax.experimental import pallas as pl
from jax.experimental.pallas import tpu as pltpu
from jax.experimental.pallas import tpu_sc as plsc

_D = 64
_CHUNK = 128
_NC = 2
_NS = 16
_NW = _NC * _NS
_NBUF = 8
_AHEAD = 4


@functools.lru_cache(maxsize=None)
def _make_gather(n_total):
    n_per_w = n_total // _NW
    n_chunks = n_per_w // _CHUNK
    assert n_chunks % _NBUF == 0
    mesh = plsc.VectorSubcoreMesh(core_axis_name="c", subcore_axis_name="s")

    @functools.partial(
        pl.kernel,
        out_type=jax.ShapeDtypeStruct((n_total, _D), jnp.float32),
        mesh=mesh,
        scratch_types=[
            pltpu.VMEM((n_chunks, _CHUNK), jnp.int32),
            pltpu.VMEM((_NBUF, _CHUNK, _D), jnp.float32),
            pltpu.SemaphoreType.DMA((_NBUF,)),
            pltpu.SemaphoreType.DMA((_NBUF,)),
        ],
        compiler_params=pltpu.CompilerParams(use_tc_tiling_on_sc=False),
    )
    def body(idx_hbm, table_hbm, out_hbm, idx_v, rows_v, g_sem, o_sem):
        wid = lax.axis_index("s") * _NC + lax.axis_index("c")
        pltpu.sync_copy(idx_hbm.at[wid], idx_v)
        base = wid * n_per_w

        def start_gather(j, slot):
            pltpu.async_copy(
                table_hbm.at[idx_v.at[j]], rows_v.at[slot], g_sem.at[slot]
            )

        for b in range(_AHEAD):
            start_gather(b, b)

        def group(g, carry):
            for b in range(_NBUF):
                j = g * _NBUF + b
                j2 = j + _AHEAD
                s2 = (b + _AHEAD) % _NBUF

                @pl.when(jnp.logical_and(j2 < n_chunks, j2 >= _NBUF))
                def _():
                    pltpu.make_async_copy(
                        rows_v.at[s2],
                        out_hbm.at[pl.ds(base, _CHUNK)],
                        o_sem.at[s2],
                    ).wait()

                @pl.when(j2 < n_chunks)
                def _():
                    start_gather(j2, s2)

                pltpu.make_async_copy(
                    table_hbm.at[idx_v.at[j]],
                    rows_v.at[b],
                    g_sem.at[b],
                ).wait()
                pltpu.async_copy(
                    rows_v.at[b],
                    out_hbm.at[pl.ds(base + j * _CHUNK, _CHUNK)],
                    o_sem.at[b],
                )
            return carry

        lax.fori_loop(0, n_chunks // _NBUF, group, 0)

        for b in range(_NBUF):
            pltpu.make_async_copy(
                rows_v.at[b], out_hbm.at[pl.ds(base, _CHUNK)], o_sem.at[b]
            ).wait()

    return body


def _make_relayout_body(hb):
    def body(flat_ref, out_ref):
        x = flat_ref[...].reshape(_CHUNK, hb * _D)
        xt = x.T
        out_ref[...] = xt.reshape(hb, 8, 1, 8, _CHUNK)

    return body


@functools.lru_cache(maxsize=None)
def _make_relayout(batch, hist):
    hb = 40
    grid = (batch // _CHUNK, hist // hb)
    return pl.pallas_call(
        _make_relayout_body(hb),
        grid=grid,
        in_specs=[
            pl.BlockSpec((_CHUNK, hb, _D), lambda i, h: (i, h, 0)),
        ],
        out_specs=pl.BlockSpec(
            (hb, 8, 1, 8, _CHUNK), lambda i, h: (h, 0, i, 0, 0)
        ),
        out_shape=jax.ShapeDtypeStruct(
            (hist, _D // 8, batch // _CHUNK, 8, _CHUNK), jnp.float32
        ),
    )


def kernel(words, table):
    b, h = words.shape
    n_total = b * h
    idx = words.reshape(_NW, n_total // _NW // _CHUNK, _CHUNK)
    flat = _make_gather(n_total)(idx, table)
    out5 = _make_relayout(b, h)(flat.reshape(b, h, _D))
    return out5.transpose(2, 4, 0, 1, 3).reshape(b, h, _D)
